# baseline (speedup 1.0000x reference)
import functools

import jax
import jax.numpy as jnp
from jax import lax
from jax.experimental import pallas as pl
from jax.experimental.pallas import tpu as pltpu
from jax.experimental.pallas import tpu_sc as plsc

B = 4096
L = 50
D = 32
N = B * L
KR = N // 128
SR = 250000
NSLAB = 7812

NC = 2
NS = 16
NW = NC * NS

RPW = KR // NW
CH = 128
NBUF = 2

_MESH = dict(core_axis_name="c", subcore_axis_name="s")


def _make_transpose():
    @functools.partial(
        pl.kernel,
        out_type=jax.ShapeDtypeStruct((SR, 128), jnp.float32),
        mesh=plsc.VectorSubcoreMesh(**_MESH),
        scratch_types=[
            pltpu.VMEM((NBUF, 32, 128), jnp.float32),
            pltpu.VMEM((NBUF, 32, 128), jnp.float32),
            pltpu.VMEM((32, 64), jnp.float32),
            pltpu.VMEM((16, 128), jnp.float32),
            [pltpu.SemaphoreType.DMA] * NBUF,
            [pltpu.SemaphoreType.DMA] * NBUF,
            pltpu.SemaphoreType.DMA,
        ],
        compiler_params=pltpu.CompilerParams(needs_layout_passes=False),
    )
    def transpose_kernel(tt_hbm, tb_hbm, slab, blk, tslab, tblk,
                         isems, osems, tsem):
        wid = lax.axis_index("s") * NC + lax.axis_index("c")
        s0 = wid * 244 + jnp.minimum(wid, 4)
        lane = jax.lax.iota(jnp.int32, 16)

        def start_read(i, buf):
            pltpu.async_copy(tt_hbm.at[:, pl.ds((s0 + i) * 128, 128)],
                             slab.at[buf], isems[buf])

        def wait_read(i, buf):
            pltpu.make_async_copy(tt_hbm.at[:, pl.ds((s0 + i) * 128, 128)],
                                  slab.at[buf], isems[buf]).wait()

        def do_transpose(src, dstref, nrows):
            def row(r, _):
                for g in range(8):
                    ivec = lane + (g % 2) * 16
                    jvec = lane * 0 + (4 * r + g // 2)
                    vals = plsc.load_gather(src, [ivec, jvec])
                    dstref[r, pl.ds(g * 16, 16)] = vals
                return _
            lax.fori_loop(0, nrows, row, ())

        def start_write(i, buf):
            pltpu.async_copy(blk.at[buf], tb_hbm.at[pl.ds((s0 + i) * 32, 32)],
                             osems[buf])

        def wait_write(i, buf):
            pltpu.make_async_copy(blk.at[buf],
                                  tb_hbm.at[pl.ds((s0 + i) * 32, 32)],
                                  osems[buf]).wait()

        for b in range(NBUF):
            start_read(b, b)

        def step(i, _):
            for b in range(NBUF):
                sl = i * NBUF + b
                wait_read(sl, b)

                @pl.when(sl >= NBUF)
                def _reclaim():
                    wait_write(sl - NBUF, b)

                do_transpose(slab.at[b], blk.at[b], 32)
                start_write(sl, b)

                @pl.when(sl + NBUF < 244)
                def _next():
                    start_read(sl + NBUF, b)
            return _

        lax.fori_loop(0, 244 // NBUF, step, ())
        wait_write(242, 0)
        wait_write(243, 1)

        @pl.when(wid < 4)
        def _extra():
            pltpu.async_copy(tt_hbm.at[:, pl.ds((s0 + 244) * 128, 128)],
                             slab.at[0], tsem).wait()
            do_transpose(slab.at[0], blk.at[0], 32)
            pltpu.async_copy(blk.at[0], tb_hbm.at[pl.ds((s0 + 244) * 32, 32)],
                             tsem).wait()

        @pl.when(wid == NW - 1)
        def _tail():
            pltpu.async_copy(tt_hbm.at[:, pl.ds(NSLAB * 128, 64)], tslab,
                             tsem).wait()
            def row(r, _):
                for g in range(8):
                    ivec = lane + (g % 2) * 16
                    jvec = lane * 0 + (4 * r + g // 2)
                    tblk[r, pl.ds(g * 16, 16)] = plsc.load_gather(
                        tslab, [ivec, jvec])
                return _
            lax.fori_loop(0, 16, row, ())
            pltpu.async_copy(tblk, tb_hbm.at[pl.ds(NSLAB * 32, 16)],
                             tsem).wait()

    return transpose_kernel


def _make_gather():
    @functools.partial(
        pl.kernel,
        out_type=jax.ShapeDtypeStruct((L, D, B), jnp.float32),
        mesh=plsc.VectorSubcoreMesh(**_MESH),
        scratch_types=[
            pltpu.VMEM((RPW, CH), jnp.int32),
            pltpu.VMEM((NBUF, CH), jnp.int32),
            pltpu.VMEM((NBUF, CH, 128), jnp.float32),
            pltpu.VMEM((NBUF, D, CH), jnp.float32),
            pltpu.VMEM((64,), jnp.int32),
            [pltpu.SemaphoreType.DMA] * NBUF,
            [pltpu.SemaphoreType.DMA] * NBUF,
            pltpu.SemaphoreType.DMA,
        ],
        compiler_params=pltpu.CompilerParams(needs_layout_passes=False),
    )
    def gather_kernel(keys_hbm, tb_hbm, out_hbm,
                      kv, sup, rows, blk, kidx, gsems, wsems, ksem):
        wid = lax.axis_index("s") * NC + lax.axis_index("c")
        kbase = wid * RPW
        lane = jax.lax.iota(jnp.int32, 16)

        for g in range(4):
            kidx[pl.ds(g * 16, 16)] = lane + (kbase + g * 16)
        pltpu.async_copy(keys_hbm.at[kidx.at[pl.ds(0, RPW)]], kv, ksem).wait()

        def start_gather(c, buf):
            for g in range(8):
                k16 = kv[c, pl.ds(g * 16, 16)]
                sup[buf, pl.ds(g * 16, 16)] = lax.shift_right_logical(k16, 2)
            pltpu.async_copy(tb_hbm.at[sup.at[buf]], rows.at[buf], gsems[buf])

        def wait_gather(buf):
            pltpu.make_async_copy(tb_hbm.at[sup.at[buf]], rows.at[buf],
                                  gsems[buf]).wait()

        def extract(c, buf):
            src = rows.at[buf]
            for g in range(8):
                k16 = kv[c, pl.ds(g * 16, 16)]
                sub16 = lax.shift_left(lax.bitwise_and(k16, 3), 5)
                ivec = lane + g * 16

                def row(d, _):
                    blk[buf, d, pl.ds(g * 16, 16)] = plsc.load_gather(
                        src, [ivec, sub16 + d])
                    return _
                lax.fori_loop(0, D, row, (), unroll=4)

        def start_write(c, buf):
            r = kbase + c
            l = r // 32
            j = r % 32
            pltpu.async_copy(blk.at[buf], out_hbm.at[l, :, pl.ds(j * 128, 128)],
                             wsems[buf])

        def wait_write(c, buf):
            r = kbase + c
            l = r // 32
            j = r % 32
            pltpu.make_async_copy(blk.at[buf],
                                  out_hbm.at[l, :, pl.ds(j * 128, 128)],
                                  wsems[buf]).wait()

        for b in range(NBUF):
            start_gather(b, b)

        def step(i, _):
            for b in range(NBUF):
                c = i * NBUF + b
                wait_gather(b)

                @pl.when(c >= NBUF)
                def _reclaim():
                    wait_write(c - NBUF, b)

                extract(c, b)
                start_write(c, b)

                @pl.when(c + NBUF < RPW)
                def _next():
                    start_gather(c + NBUF, b)
            return _

        lax.fori_loop(0, RPW // NBUF, step, ())
        wait_write(RPW - 2, 0)
        wait_write(RPW - 1, 1)

    return gather_kernel


_transpose = _make_transpose()
_gather = _make_gather()


@jax.jit
def kernel(keys, table):
    tt = table.T
    keys_r = keys.astype(jnp.int32).T.reshape((KR, 128))
    tb = _transpose(tt)
    out3 = _gather(keys_r, tb)
    return out3.transpose((2, 0, 1))


# speedup vs baseline: 1.2061x; 1.2061x over previous
import functools

import jax
import jax.numpy as jnp
from jax import lax
from jax.experimental import pallas as pl
from jax.experimental.pallas import tpu as pltpu
from jax.experimental.pallas import tpu_sc as plsc

B = 4096
L = 50
D = 32
N = B * L
KR = N // 128
SR = 250000
NSLAB = 7812

NC = 2
NS = 16
NW = NC * NS

RPW = KR // NW
CH = 128
NBUF = 2

_MESH = dict(core_axis_name="c", subcore_axis_name="s")
_PARAMS = pltpu.CompilerParams(needs_layout_passes=False)


def _make_transpose():
    @functools.partial(
        pl.kernel,
        out_type=jax.ShapeDtypeStruct((SR, 128), jnp.float32),
        mesh=plsc.VectorSubcoreMesh(**_MESH),
        scratch_types=[
            pltpu.VMEM((NBUF, 32, 128), jnp.float32),
            pltpu.VMEM((NBUF, 32, 128), jnp.float32),
            pltpu.VMEM((32, 64), jnp.float32),
            pltpu.VMEM((16, 128), jnp.float32),
            [pltpu.SemaphoreType.DMA] * NBUF,
            [pltpu.SemaphoreType.DMA] * NBUF,
            pltpu.SemaphoreType.DMA,
        ],
        compiler_params=_PARAMS,
    )
    def transpose_kernel(tt_hbm, tb_hbm, slab, blk, tslab, tblk,
                         isems, osems, tsem):
        wid = lax.axis_index("s") * NC + lax.axis_index("c")
        s0 = wid * 244 + jnp.minimum(wid, 4)
        lane = jax.lax.iota(jnp.int32, 16)

        def start_read(i, buf):
            pltpu.async_copy(tt_hbm.at[:, pl.ds((s0 + i) * 128, 128)],
                             slab.at[buf], isems[buf])

        def wait_read(i, buf):
            pltpu.make_async_copy(tt_hbm.at[:, pl.ds((s0 + i) * 128, 128)],
                                  slab.at[buf], isems[buf]).wait()

        def do_transpose(src, dstref, ncols):
            jbase = (lane % 4) * 32
            for f in range(32):
                jvec = jbase + f
                for cb in range(ncols // 16):
                    ivec = (lane + cb * 16) // 4
                    x = src[f, pl.ds(cb * 16, 16)]
                    plsc.store_scatter(dstref, [ivec, jvec], x)

        def start_write(i, buf):
            pltpu.async_copy(
                blk.at[buf],
                tb_hbm.at[pl.ds((s0 + i) * 32, 32)],
                osems[buf])

        def wait_write(i, buf):
            pltpu.make_async_copy(
                blk.at[buf],
                tb_hbm.at[pl.ds((s0 + i) * 32, 32)],
                osems[buf]).wait()

        for b in range(NBUF):
            start_read(b, b)

        def step(i, _):
            for b in range(NBUF):
                sl = i * NBUF + b
                wait_read(sl, b)

                @pl.when(sl >= NBUF)
                def _reclaim():
                    wait_write(sl - NBUF, b)

                do_transpose(slab.at[b], blk.at[b], 128)
                start_write(sl, b)

                @pl.when(sl + NBUF < 244)
                def _next():
                    start_read(sl + NBUF, b)
            return _

        lax.fori_loop(0, 244 // NBUF, step, ())
        wait_write(242, 0)
        wait_write(243, 1)

        @pl.when(wid < 4)
        def _extra():
            pltpu.async_copy(tt_hbm.at[:, pl.ds((s0 + 244) * 128, 128)],
                             slab.at[0], tsem).wait()
            do_transpose(slab.at[0], blk.at[0], 128)
            pltpu.async_copy(
                blk.at[0], tb_hbm.at[pl.ds((s0 + 244) * 32, 32)],
                tsem).wait()

        @pl.when(wid == NW - 1)
        def _tail():
            pltpu.async_copy(tt_hbm.at[:, pl.ds(NSLAB * 128, 64)], tslab,
                             tsem).wait()
            do_transpose(tslab, tblk, 64)
            pltpu.async_copy(
                tblk, tb_hbm.at[pl.ds(NSLAB * 32, 16)], tsem).wait()

    return transpose_kernel


def _make_gather():
    @functools.partial(
        pl.kernel,
        out_type=jax.ShapeDtypeStruct((L, D, B), jnp.float32),
        mesh=plsc.VectorSubcoreMesh(**_MESH),
        scratch_types=[
            pltpu.VMEM((RPW, CH), jnp.int32),
            pltpu.VMEM((NBUF, CH), jnp.int32),
            pltpu.VMEM((NBUF, CH, 128), jnp.float32),
            pltpu.VMEM((NBUF, D, CH), jnp.float32),
            pltpu.VMEM((64,), jnp.int32),
            [pltpu.SemaphoreType.DMA] * NBUF,
            [pltpu.SemaphoreType.DMA] * NBUF,
            pltpu.SemaphoreType.DMA,
        ],
        compiler_params=_PARAMS,
    )
    def gather_kernel(keys_hbm, tb_hbm, out_hbm,
                      kv, sup, rows, blk, kidx, gsems, wsems, ksem):
        wid = lax.axis_index("s") * NC + lax.axis_index("c")
        kbase = wid * RPW
        lane = jax.lax.iota(jnp.int32, 16)

        for g in range(4):
            kidx[pl.ds(g * 16, 16)] = lane + (kbase + g * 16)
        pltpu.async_copy(keys_hbm.at[kidx.at[pl.ds(0, RPW)]], kv, ksem).wait()

        def start_gather(c, buf):
            for g in range(8):
                k16 = kv[c, pl.ds(g * 16, 16)]
                sup[buf, pl.ds(g * 16, 16)] = lax.shift_right_logical(k16, 2)
            pltpu.async_copy(tb_hbm.at[sup.at[buf]], rows.at[buf], gsems[buf])

        def wait_gather(c, buf):
            pltpu.make_async_copy(tb_hbm.at[sup.at[buf]], rows.at[buf],
                                  gsems[buf]).wait()

        def extract(c, buf):
            src = rows.at[buf]
            dstref = blk.at[buf]

            def grp(g, _):
                k16 = kv[c, pl.ds(g * 16, 16)]
                sub16 = lax.shift_left(lax.bitwise_and(k16, 3), 5)
                for t in range(16):
                    kk = g * 16 + t
                    s = sub16[t]
                    kvec = lane * 0 + kk
                    for v in range(2):
                        x = src[kk, pl.ds(s + v * 16, 16)]
                        plsc.store_scatter(dstref, [lane + v * 16, kvec], x)
                return _
            lax.fori_loop(0, CH // 16, grp, ())

        def start_write(c, buf):
            r = kbase + c
            pltpu.async_copy(
                blk.at[buf],
                out_hbm.at[r // 32, :, pl.ds((r % 32) * 128, 128)],
                wsems[buf])

        def wait_write(c, buf):
            r = kbase + c
            pltpu.make_async_copy(
                blk.at[buf],
                out_hbm.at[r // 32, :, pl.ds((r % 32) * 128, 128)],
                wsems[buf]).wait()

        for b in range(NBUF):
            start_gather(b, b)

        def step(i, _):
            for b in range(NBUF):
                c = i * NBUF + b
                wait_gather(c, b)

                @pl.when(c >= NBUF)
                def _reclaim():
                    wait_write(c - NBUF, b)

                extract(c, b)
                start_write(c, b)

                @pl.when(c + NBUF < RPW)
                def _next():
                    start_gather(c + NBUF, b)
            return _

        lax.fori_loop(0, RPW // NBUF, step, ())
        wait_write(RPW - 2, 0)
        wait_write(RPW - 1, 1)

    return gather_kernel


_transpose = _make_transpose()
_gather = _make_gather()


@jax.jit
def kernel(keys, table):
    tt = table.T
    keys_r = keys.astype(jnp.int32).T.reshape((KR, 128))
    tb = _transpose(tt)
    out3 = _gather(keys_r, tb)
    return out3.transpose((2, 0, 1))


# speedup vs baseline: 1.5714x; 1.3028x over previous
import functools

import jax
import jax.numpy as jnp
from jax import lax
from jax.experimental import pallas as pl
from jax.experimental.pallas import tpu as pltpu
from jax.experimental.pallas import tpu_sc as plsc

B = 4096
L = 50
D = 32
N = B * L
KR = N // 128
SR = 250000
NSLAB = 7812

NC = 2
NS = 16
NW = NC * NS

RPW = KR // NW
CH = 128
NBUF = 2

_MESH = dict(core_axis_name="c", subcore_axis_name="s")
_PARAMS = pltpu.CompilerParams(needs_layout_passes=False)


def _make_transpose():
    @functools.partial(
        pl.kernel,
        out_type=jax.ShapeDtypeStruct((SR, 128), jnp.float32),
        mesh=plsc.VectorSubcoreMesh(**_MESH),
        scratch_types=[
            pltpu.VMEM((NBUF, 32, 128), jnp.float32),
            pltpu.VMEM((NBUF, 32, 128), jnp.float32),
            pltpu.VMEM((32, 64), jnp.float32),
            pltpu.VMEM((16, 128), jnp.float32),
            [pltpu.SemaphoreType.DMA] * NBUF,
            [pltpu.SemaphoreType.DMA] * NBUF,
            pltpu.SemaphoreType.DMA,
        ],
        compiler_params=_PARAMS,
    )
    def transpose_kernel(tt_hbm, tb_hbm, slab, blk, tslab, tblk,
                         isems, osems, tsem):
        wid = lax.axis_index("s") * NC + lax.axis_index("c")
        s0 = wid * 244 + jnp.minimum(wid, 4)
        lane = jax.lax.iota(jnp.int32, 16)

        def start_read(i, buf):
            pltpu.async_copy(tt_hbm.at[:, pl.ds((s0 + i) * 128, 128)],
                             slab.at[buf], isems[buf])

        def wait_read(i, buf):
            pltpu.make_async_copy(tt_hbm.at[:, pl.ds((s0 + i) * 128, 128)],
                                  slab.at[buf], isems[buf]).wait()

        def do_transpose(src, dstref, ncols):
            for fr in range(2):
                src_i = lane + 16 * fr
                for cb in range(ncols // 16):
                    for k in range(16):
                        dr = lax.bitwise_and(lane + k, 15)
                        x = plsc.load_gather(src, [src_i, dr + 16 * cb])
                        dst_i = (dr // 4) + 4 * cb
                        dst_j = lax.rem(dr, 4) * 32 + src_i
                        plsc.store_scatter(dstref, [dst_i, dst_j], x)

        def start_write(i, buf):
            pltpu.async_copy(
                blk.at[buf],
                tb_hbm.at[pl.ds((s0 + i) * 32, 32)],
                osems[buf])

        def wait_write(i, buf):
            pltpu.make_async_copy(
                blk.at[buf],
                tb_hbm.at[pl.ds((s0 + i) * 32, 32)],
                osems[buf]).wait()

        for b in range(NBUF):
            start_read(b, b)

        def step(i, _):
            for b in range(NBUF):
                sl = i * NBUF + b
                wait_read(sl, b)

                @pl.when(sl >= NBUF)
                def _reclaim():
                    wait_write(sl - NBUF, b)

                do_transpose(slab.at[b], blk.at[b], 128)
                start_write(sl, b)

                @pl.when(sl + NBUF < 244)
                def _next():
                    start_read(sl + NBUF, b)
            return _

        lax.fori_loop(0, 244 // NBUF, step, ())
        wait_write(242, 0)
        wait_write(243, 1)

        @pl.when(wid < 4)
        def _extra():
            pltpu.async_copy(tt_hbm.at[:, pl.ds((s0 + 244) * 128, 128)],
                             slab.at[0], tsem).wait()
            do_transpose(slab.at[0], blk.at[0], 128)
            pltpu.async_copy(
                blk.at[0], tb_hbm.at[pl.ds((s0 + 244) * 32, 32)],
                tsem).wait()

        @pl.when(wid == NW - 1)
        def _tail():
            pltpu.async_copy(tt_hbm.at[:, pl.ds(NSLAB * 128, 64)], tslab,
                             tsem).wait()
            do_transpose(tslab, tblk, 64)
            pltpu.async_copy(
                tblk, tb_hbm.at[pl.ds(NSLAB * 32, 16)], tsem).wait()

    return transpose_kernel


def _make_gather():
    @functools.partial(
        pl.kernel,
        out_type=jax.ShapeDtypeStruct((L, D, B), jnp.float32),
        mesh=plsc.VectorSubcoreMesh(**_MESH),
        scratch_types=[
            pltpu.VMEM((RPW, CH), jnp.int32),
            pltpu.VMEM((NBUF, CH), jnp.int32),
            pltpu.VMEM((NBUF, CH, 128), jnp.float32),
            pltpu.VMEM((NBUF, D, CH), jnp.float32),
            pltpu.VMEM((64,), jnp.int32),
            [pltpu.SemaphoreType.DMA] * NBUF,
            [pltpu.SemaphoreType.DMA] * NBUF,
            pltpu.SemaphoreType.DMA,
        ],
        compiler_params=_PARAMS,
    )
    def gather_kernel(keys_hbm, tb_hbm, out_hbm,
                      kv, sup, rows, blk, kidx, gsems, wsems, ksem):
        wid = lax.axis_index("s") * NC + lax.axis_index("c")
        kbase = wid * RPW
        lane = jax.lax.iota(jnp.int32, 16)

        for g in range(4):
            kidx[pl.ds(g * 16, 16)] = lane + (kbase + g * 16)
        pltpu.async_copy(keys_hbm.at[kidx.at[pl.ds(0, RPW)]], kv, ksem).wait()

        def start_gather(c, buf):
            for g in range(8):
                k16 = kv[c, pl.ds(g * 16, 16)]
                sup[buf, pl.ds(g * 16, 16)] = lax.shift_right_logical(k16, 2)
            pltpu.async_copy(tb_hbm.at[sup.at[buf]], rows.at[buf], gsems[buf])

        def wait_gather(c, buf):
            pltpu.make_async_copy(tb_hbm.at[sup.at[buf]], rows.at[buf],
                                  gsems[buf]).wait()

        def extract(c, buf):
            src = rows.at[buf]
            dstref = blk.at[buf]

            def grp(g, _):
                k16 = kv[c, pl.ds(g * 16, 16)]
                sub16 = lax.shift_left(lax.bitwise_and(k16, 3), 5)
                src_i = lane + g * 16
                for v in range(2):
                    for k in range(16):
                        dr = lax.bitwise_and(lane + k, 15)
                        x = plsc.load_gather(src, [src_i, sub16 + 16 * v + dr])
                        plsc.store_scatter(dstref, [16 * v + dr, src_i], x)
                return _
            lax.fori_loop(0, CH // 16, grp, ())

        def start_write(c, buf):
            r = kbase + c
            pltpu.async_copy(
                blk.at[buf],
                out_hbm.at[r // 32, :, pl.ds((r % 32) * 128, 128)],
                wsems[buf])

        def wait_write(c, buf):
            r = kbase + c
            pltpu.make_async_copy(
                blk.at[buf],
                out_hbm.at[r // 32, :, pl.ds((r % 32) * 128, 128)],
                wsems[buf]).wait()

        for b in range(NBUF):
            start_gather(b, b)

        def step(i, _):
            for b in range(NBUF):
                c = i * NBUF + b
                wait_gather(c, b)

                @pl.when(c >= NBUF)
                def _reclaim():
                    wait_write(c - NBUF, b)

                extract(c, b)
                start_write(c, b)

                @pl.when(c + NBUF < RPW)
                def _next():
                    start_gather(c + NBUF, b)
            return _

        lax.fori_loop(0, RPW // NBUF, step, ())
        wait_write(RPW - 2, 0)
        wait_write(RPW - 1, 1)

    return gather_kernel


_transpose = _make_transpose()
_gather = _make_gather()


@jax.jit
def kernel(keys, table):
    tt = table.T
    keys_r = keys.astype(jnp.int32).T.reshape((KR, 128))
    tb = _transpose(tt)
    out3 = _gather(keys_r, tb)
    return out3.transpose((2, 0, 1))
